# Initial kernel scaffold; baseline (speedup 1.0000x reference)
import jax
import jax.numpy as jnp
from jax.experimental import pallas as pl


def kernel(output, target):
    raise NotImplementedError("write your pallas kernel here")


# speedup vs baseline: 12.3473x; 12.3473x over previous
import functools

import jax
import jax.numpy as jnp
from jax.experimental import pallas as pl
from jax.experimental.pallas import tpu as pltpu

_N = 8 * 512 * 512
_COLS = 1024
_ROWS = _N // _COLS
_BLK_ROWS = 256
_GRID = _ROWS // _BLK_ROWS
_BIG = 2**30


def _ohem_body(x0_ref, x1_ref, tg_ref, out_ref, fs_ref, is_ref):
    pid = pl.program_id(0)

    @pl.when(pid == 0)
    def _init():
        fs_ref[0] = jnp.float32(-jnp.inf)
        fs_ref[1] = jnp.float32(-jnp.inf)
        for i in range(2, 10):
            fs_ref[i] = jnp.float32(0.0)
        fs_ref[10] = x0_ref[0, 0]
        fs_ref[11] = x1_ref[0, 0]
        is_ref[0] = jnp.int32(0)
        for i in range(1, 5):
            is_ref[i] = _BIG

    x0 = x0_ref[...]
    x1 = x1_ref[...]
    m = tg_ref[...] == 1

    m0_old, m1_old = fs_ref[0], fs_ref[1]
    m0 = jnp.maximum(m0_old, jnp.max(x0))
    m1 = jnp.maximum(m1_old, jnp.max(x1))
    r0 = jnp.exp(m0_old - m0)
    r1 = jnp.exp(m1_old - m1)
    e0 = jnp.exp(x0 - m0)
    e1 = jnp.exp(x1 - m1)
    zf = jnp.float32(0.0)
    fs_ref[0] = m0
    fs_ref[1] = m1
    fs_ref[2] = fs_ref[2] * r0 + jnp.sum(jnp.where(m, e0, zf))
    fs_ref[3] = fs_ref[3] * r0 + jnp.sum(jnp.where(m, zf, e0))
    fs_ref[4] = fs_ref[4] * r1 + jnp.sum(jnp.where(m, e1, zf))
    fs_ref[5] = fs_ref[5] * r1 + jnp.sum(jnp.where(m, zf, e1))

    cnt_before = is_ref[0]
    blk_pos = jnp.sum(m.astype(jnp.int32))
    is_ref[0] = cnt_before + blk_pos
    blk_elems = jnp.int32(_BLK_ROWS * _COLS)

    def _track(mask, g1_i, g2_i, v0_i, v1_i, cnt_b):
        @pl.when(cnt_b < 2)
        def _():
            off = pid * blk_elems
            ri = jax.lax.broadcasted_iota(jnp.int32, (_BLK_ROWS, _COLS), 0)
            ci = jax.lax.broadcasted_iota(jnp.int32, (_BLK_ROWS, _COLS), 1)
            gidx = off + ri * jnp.int32(_COLS) + ci
            li = jnp.where(mask, gidx, _BIG)
            c1 = jnp.min(li)
            c2 = jnp.min(jnp.where(li == c1, _BIG, li))
            g1 = is_ref[g1_i]
            new_g2 = jnp.where(g1 < _BIG, c1, c2)
            sel = li == jnp.where(new_g2 == c1, c1, c2)
            v0 = jnp.sum(jnp.where(sel, x0, zf))
            v1 = jnp.sum(jnp.where(sel, x1, zf))
            g2 = is_ref[g2_i]
            take = (g2 >= _BIG) & (new_g2 < _BIG)
            is_ref[g1_i] = jnp.minimum(g1, c1)
            is_ref[g2_i] = jnp.where(take, new_g2, g2)
            fs_ref[v0_i] = jnp.where(take, v0, fs_ref[v0_i])
            fs_ref[v1_i] = jnp.where(take, v1, fs_ref[v1_i])

    _track(m, 1, 2, 6, 7, cnt_before)
    _track(jnp.logical_not(m), 3, 4, 8, 9,
           pid * blk_elems - cnt_before)

    @pl.when(pid == _GRID - 1)
    def _fin():
        lse0p = fs_ref[0] + jnp.log(fs_ref[2])
        lse0n = fs_ref[0] + jnp.log(fs_ref[3])
        lse1p = fs_ref[1] + jnp.log(fs_ref[4])
        lse1n = fs_ref[1] + jnp.log(fs_ref[5])
        vp0 = jnp.where(is_ref[2] < _BIG, fs_ref[6], fs_ref[10])
        vp1 = jnp.where(is_ref[2] < _BIG, fs_ref[7], fs_ref[11])
        vn0 = jnp.where(is_ref[4] < _BIG, fs_ref[8], fs_ref[10])
        vn1 = jnp.where(is_ref[4] < _BIG, fs_ref[9], fs_ref[11])
        pos_losses = 0.5 * ((lse0p - vp0) + (lse1p - vp1))
        neg0 = lse0n - vn0
        neg1 = lse1n - vn1
        npos = is_ref[0]
        k = jnp.minimum(6 * npos, 2)
        hi = jnp.maximum(neg0, neg1)
        lo = jnp.minimum(neg0, neg1)
        s = jnp.where(k >= 1, hi, zf) + jnp.where(k >= 2, lo, zf)
        neg_topk = s / k.astype(jnp.float32)
        out_ref[0, 0] = (neg_topk + 3.0 * pos_losses) * 0.25


@functools.partial(jax.jit, static_argnames=("interpret",))
def _ohem(x0, x1, tg, interpret=False):
    return pl.pallas_call(
        _ohem_body,
        grid=(_GRID,),
        in_specs=[
            pl.BlockSpec((_BLK_ROWS, _COLS), lambda i: (i, 0)),
            pl.BlockSpec((_BLK_ROWS, _COLS), lambda i: (i, 0)),
            pl.BlockSpec((_BLK_ROWS, _COLS), lambda i: (i, 0)),
        ],
        out_specs=pl.BlockSpec(memory_space=pltpu.SMEM),
        out_shape=jax.ShapeDtypeStruct((1, 1), jnp.float32),
        scratch_shapes=[
            pltpu.SMEM((12,), jnp.float32),
            pltpu.SMEM((8,), jnp.int32),
        ],
        interpret=interpret,
    )(x0, x1, tg)


def kernel(output, target):
    x0 = output[:, 0, :, :].reshape(_ROWS, _COLS)
    x1 = output[:, 1, :, :].reshape(_ROWS, _COLS)
    tg = target.reshape(_ROWS, _COLS)
    return _ohem(x0, x1, tg)[0, 0]
